# baseline (speedup 1.0000x reference)
import functools

import jax
import jax.numpy as jnp
from jax import lax
from jax.experimental import pallas as pl
from jax.experimental.pallas import tpu as pltpu

_NUM_CLASSES = 80
_IMG_SIZE = 1024.0
_LAMBDA_COORD = 5.0
_LAMBDA_NOOBJ = 0.5


def _softplus(z):
    return jnp.maximum(z, 0.0) + jnp.log1p(jnp.exp(-jnp.abs(z)))


def _tree_reduce(mats, op):
    while len(mats) > 1:
        nxt = [op(mats[i], mats[i + 1]) for i in range(0, len(mats) - 1, 2)]
        if len(mats) % 2:
            nxt.append(mats[-1])
        mats = nxt
    return mats[0]


def _block_kernel(anc_ref, pred_ref, tgt_ref, out_ref, sel_ref, *, na, nai, nattr, r, g):
    lanes = 6 * g

    @pl.when(pl.program_id(0) == 0)
    def _build_sel():
        row = lax.broadcasted_iota(jnp.int32, (lanes, lanes), 0)
        col = lax.broadcasted_iota(jnp.int32, (lanes, lanes), 1)
        want = (row % 6) * g + row // 6
        sel_ref[...] = (want == col).astype(jnp.bfloat16)

    tmat = tgt_ref[...].reshape(na * r, lanes)
    t_hi = tmat.astype(jnp.bfloat16)
    t_lo = (tmat - t_hi.astype(jnp.float32)).astype(jnp.bfloat16)
    sel = sel_ref[...]
    dims = (((1,), (0,)), ((), ()))
    d = lax.dot_general(
        t_hi, sel, dims, preferred_element_type=jnp.float32
    ) + lax.dot_general(t_lo, sel, dims, preferred_element_type=jnp.float32)

    def attr(k):
        return d[:, k * g : (k + 1) * g].reshape(na, r, g)

    tx, ty, tw, th, t4, t5 = (attr(k) for k in range(6))
    obj = (t4 > 0.0).astype(jnp.float32)
    noobj = (t4 == 0.0).astype(jnp.float32)
    n_obj = jnp.sum(obj)
    n_noobj = jnp.sum(noobj)

    def rows(a0):
        return jnp.concatenate(
            [pred_ref[0, a * nattr + a0 : a * nattr + a0 + 1] for a in range(na)],
            axis=0,
        )

    x = jax.nn.sigmoid(rows(0))
    sx = jnp.sum(obj * (x - tx) ** 2)
    y = jax.nn.sigmoid(rows(1))
    sy = jnp.sum(obj * (y - ty) ** 2)

    ew = jnp.exp(rows(2))
    eh = jnp.exp(rows(3))
    ew2 = ew * ew
    eh2 = eh * eh

    def weighted(mat, coef):
        outs = []
        for i in range(na):
            acc = coef(i, 0) * mat[0:1]
            for j in range(1, nai):
                acc = acc + coef(i, j) * mat[j : j + 1]
            outs.append(acc)
        return jnp.concatenate(outs, axis=0)

    aw = lambda i, j: anc_ref[0, i, j]
    ah = lambda i, j: anc_ref[1, i, j]
    aw2 = lambda i, j: anc_ref[0, i, j] * anc_ref[0, i, j]
    ah2 = lambda i, j: anc_ref[1, i, j] * anc_ref[1, i, j]
    fnai = float(nai)
    sw = jnp.sum(
        obj * (weighted(ew2, aw2) - 2.0 * tw * weighted(ew, aw) + fnai * tw * tw)
    )
    sh = jnp.sum(
        obj * (weighted(eh2, ah2) - 2.0 * th * weighted(eh, ah) + fnai * th * th)
    )

    zc = rows(4)
    ec = _softplus(zc) - t4 * zc
    so = jnp.sum(obj * ec)
    sn = jnp.sum(noobj * ec)

    sc = jnp.float32(0.0)
    for a in range(na):
        sum_z = None
        sum_az = None
        prod_w = None
        chunk = 8
        for c0 in range(5, nattr, chunk):
            zs = [pred_ref[0, a * nattr + c] for c in range(c0, min(c0 + chunk, nattr))]
            azs = [jnp.abs(z) for z in zs]
            ws = [1.0 + jnp.exp(-az) for az in azs]
            pz = _tree_reduce(zs, jnp.add)
            paz = _tree_reduce(azs, jnp.add)
            pw = _tree_reduce(ws, jnp.multiply)
            sum_z = pz if sum_z is None else sum_z + pz
            sum_az = paz if sum_az is None else sum_az + paz
            prod_w = pw if prod_w is None else prod_w * pw
        s_sum = 0.5 * (sum_z + sum_az) + jnp.log(prod_w)
        sc = sc + jnp.sum(obj[a] * (s_sum - t5[a] * sum_z))

    vals = [n_obj, n_noobj, sx, sy, sw, sh, so, sn, sc]
    for k, v in enumerate(vals):
        out_ref[0, 0, k] = v
    for k in range(len(vals), 16):
        out_ref[0, 0, k] = jnp.float32(0.0)


def kernel(predictions, targets, anchors):
    b, ch, g, g2 = predictions.shape
    na = targets.shape[1]
    nattr = ch // na
    nai = anchors.shape[1]
    stride = _IMG_SIZE / g
    scaled = anchors / stride
    anc = jnp.stack([scaled[:, :, 0], scaled[:, :, 1]])

    tgt = targets.reshape(na, g, g2 * 6)

    nblocks = 8
    r = g // nblocks

    body = functools.partial(
        _block_kernel, na=na, nai=nai, nattr=nattr, r=r, g=g2
    )
    partials = pl.pallas_call(
        body,
        grid=(nblocks,),
        in_specs=[
            pl.BlockSpec(memory_space=pltpu.SMEM),
            pl.BlockSpec((1, ch, r, g2), lambda i: (0, 0, i, 0)),
            pl.BlockSpec((na, r, g2 * 6), lambda i: (0, i, 0)),
        ],
        out_specs=pl.BlockSpec(
            (1, 1, 16), lambda i: (i, 0, 0), memory_space=pltpu.SMEM
        ),
        out_shape=jax.ShapeDtypeStruct((nblocks, 1, 16), jnp.float32),
        scratch_shapes=[pltpu.VMEM((g2 * 6, g2 * 6), jnp.bfloat16)],
    )(anc, predictions, tgt)

    p = jnp.sum(partials.reshape(nblocks, 16), axis=0)
    n_obj, n_noobj = p[0], p[1]
    sx, sy, sw, sh, so, sn, sc = p[2], p[3], p[4], p[5], p[6], p[7], p[8]
    total = (
        (_LAMBDA_COORD * (sx + sy) + sw + sh + so) / n_obj
        + _LAMBDA_NOOBJ * sn / n_noobj
        + sc / (n_obj * _NUM_CLASSES)
    )
    return total
